# baseline (speedup 1.0000x reference)
import functools

import jax
import jax.numpy as jnp
from jax import lax
from jax.experimental import pallas as pl
from jax.experimental.pallas import tpu as pltpu
from jax.experimental.pallas import tpu_sc as plsc

V1 = 1000001
H = 64
B = 16384
L = 200


_CBLK = 16384
_NBLK = (V1 + _CBLK - 1) // _CBLK
_ROWS_PAD = _NBLK * _CBLK
_PW = 32


def _transform_body(t_ref, w_ref, o_ref):
    r = lax.dot_general(
        t_ref[...], w_ref[...],
        dimension_numbers=(((0,), (0,)), ((), ())),
        preferred_element_type=jnp.float32,
    )
    a = lax.bitcast_convert_type(r[:, 0:128], jnp.uint32) >> 16
    bvals = lax.bitcast_convert_type(r[:, 128:256], jnp.uint32) >> 16
    w = a | (bvals << 16)
    s = w & jnp.uint32(0x80008000)
    m = s - (s >> 15)
    o_ref[...] = w ^ m ^ jnp.uint32(0x80008000)


def _transform_table(table, W):
    table_t = table.T
    wt = W.T
    cols = jnp.arange(2 * H) % _PW
    wab = jnp.concatenate([wt[:, cols], wt[:, cols + _PW]], axis=1)
    out = pl.pallas_call(
        _transform_body,
        grid=(_NBLK,),
        in_specs=[
            pl.BlockSpec((H, _CBLK), lambda i: (0, i)),
            pl.BlockSpec((H, 4 * H), lambda i: (0, 0)),
        ],
        out_specs=pl.BlockSpec((_CBLK, 2 * H), lambda i: (i, 0)),
        out_shape=jax.ShapeDtypeStruct((_ROWS_PAD, 2 * H), jnp.uint32),
    )(table_t, wab)
    return out.reshape(4 * _ROWS_PAD, _PW)


_G = 8
_CHUNK = 100
_NW = 32
_GROUPS = B // _G
_GPW = _GROUPS // _NW
_OUTBUF_GROUPS = 8


def _sc_mesh():
    return plsc.VectorSubcoreMesh(core_axis_name="c", subcore_axis_name="s")


@functools.partial(
    pl.kernel,
    out_type=jax.ShapeDtypeStruct((B, _PW), jnp.uint32),
    mesh=_sc_mesh(),
    compiler_params=pltpu.CompilerParams(use_tc_tiling_on_sc=False),
    scratch_types=[
        pltpu.VMEM((2, _G * L), jnp.int32),
        pltpu.VMEM((2, _G * L, _PW), jnp.uint32),
        pltpu.VMEM((_OUTBUF_GROUPS * _G, _PW), jnp.uint32),
        pltpu.SemaphoreType.DMA,
        pltpu.SemaphoreType.DMA,
        pltpu.SemaphoreType.DMA,
        pltpu.SemaphoreType.DMA,
    ],
)
def _sc_gather_max(x_hbm, table2_hbm, out_hbm,
                   idx_v, rows_v, out_v,
                   isem0, isem1, gsem0, gsem1):
    nc = 2
    wid = lax.axis_index("s") * nc + lax.axis_index("c")
    g_base = wid * _GPW
    g_last = g_base + _GPW - 1
    isems = (isem0, isem1)
    gsems = (gsem0, gsem1)

    def fire_idx(bb, g):
        pltpu.async_copy(x_hbm.at[pl.ds(g * (_G * L), _G * L)],
                         idx_v.at[bb, pl.ds(0, _G * L)], isems[bb])

    def wait_idx(bb):
        pltpu.make_async_copy(x_hbm.at[pl.ds(g_base * (_G * L), _G * L)],
                              idx_v.at[bb, pl.ds(0, _G * L)],
                              isems[bb]).wait()

    def _chunks(bb):
        total = _G * L
        for off in range(0, total, 128):
            n = min(128, total - off)
            yield (idx_v.at[bb, pl.ds(off, n)],
                   rows_v.at[bb, pl.ds(off, n)])

    def fire_gather(bb):
        wait_idx(bb)
        for k in range(_G * L // 16):
            sl = pl.ds(k * 16, 16)
            idx_v[bb, sl] = idx_v[bb, sl] << 2
        for idx_ref, dst in _chunks(bb):
            pltpu.async_copy(table2_hbm.at[idx_ref], dst, gsems[bb])

    def drain_gather(bb):
        for idx_ref, dst in _chunks(bb):
            pltpu.make_async_copy(table2_hbm.at[idx_ref], dst,
                                  gsems[bb]).wait()

    def reduce_store(bb, slot):
        def red_body(j, acc):
            new = []
            for s in range(_G):
                row = s * L + j
                for h in range(2):
                    v = rows_v[bb, row, pl.ds(h * 16, 16)]
                    new.append(jnp.maximum(acc[s * 4 + 2 * h], v))
                    new.append(jnp.maximum(acc[s * 4 + 2 * h + 1], v << 16))
            return tuple(new)

        zero = jnp.zeros((16,), jnp.uint32)
        init = tuple(zero for _ in range(_G * 4))
        acc = lax.fori_loop(0, L, red_body, init)
        for s in range(_G):
            row = slot * _G + s
            for h in range(2):
                hi_bits = acc[s * 4 + 2 * h]
                lo_bits = acc[s * 4 + 2 * h + 1]
                word = (hi_bits & jnp.uint32(0xFFFF0000)) | (lo_bits >> 16)
                out_v[row, pl.ds(h * 16, 16)] = word

    fire_idx(0, g_base)
    fire_idx(1, g_base + 1)
    fire_gather(0)

    def body(i2, _):
        ga = g_base + 2 * i2
        gc = jnp.minimum(ga + 2, g_last)
        gd = jnp.minimum(ga + 3, g_last)
        m = lax.rem(i2, 4)

        fire_gather(1)
        drain_gather(0)
        fire_idx(0, gc)
        reduce_store(0, m * 2)
        fire_gather(0)
        drain_gather(1)
        fire_idx(1, gd)
        reduce_store(1, m * 2 + 1)

        @pl.when(m == 3)
        def _flush():
            row0 = (ga - 6) * _G
            pltpu.sync_copy(
                out_v, out_hbm.at[pl.ds(row0, _OUTBUF_GROUPS * _G)])

        return ()

    lax.fori_loop(0, _GPW // 2, body, ())
    drain_gather(0)
    wait_idx(1)


def kernel(x, table, W, b):
    table2 = _transform_table(table, W)
    x1 = x.astype(jnp.int32).reshape(B * L)
    out_pack = _sc_gather_max(x1, table2)
    def _unkey(k16):
        bits = jnp.where(k16 >= jnp.uint32(0x8000),
                         k16 ^ jnp.uint32(0x8000),
                         k16 ^ jnp.uint32(0xFFFF))
        return lax.bitcast_convert_type(bits << 16, jnp.float32)

    flo = _unkey(out_pack & jnp.uint32(0xFFFF))
    fhi = _unkey(out_pack >> 16)
    return jnp.concatenate([flo, fhi], axis=1) + b


# speedup vs baseline: 1.0012x; 1.0012x over previous
import functools

import jax
import jax.numpy as jnp
from jax import lax
from jax.experimental import pallas as pl
from jax.experimental.pallas import tpu as pltpu
from jax.experimental.pallas import tpu_sc as plsc

V1 = 1000001
H = 64
B = 16384
L = 200


_CBLK = 16384
_NBLK = (V1 + _CBLK - 1) // _CBLK
_ROWS_PAD = _NBLK * _CBLK
_PW = 32


def _transform_body(t_ref, w_ref, o_ref):
    r = lax.dot_general(
        t_ref[...], w_ref[...],
        dimension_numbers=(((0,), (0,)), ((), ())),
        preferred_element_type=jnp.float32,
    )
    a = lax.bitcast_convert_type(r[:, 0:128], jnp.uint32) >> 16
    bvals = lax.bitcast_convert_type(r[:, 128:256], jnp.uint32) >> 16
    w = a | (bvals << 16)
    s = w & jnp.uint32(0x80008000)
    m = s - (s >> 15)
    o_ref[...] = w ^ m ^ jnp.uint32(0x80008000)


def _transform_table(table, W):
    table_t = table.T
    wt = W.T
    cols = jnp.arange(2 * H) % _PW
    wab = jnp.concatenate([wt[:, cols], wt[:, cols + _PW]], axis=1)
    out = pl.pallas_call(
        _transform_body,
        grid=(_NBLK,),
        in_specs=[
            pl.BlockSpec((H, _CBLK), lambda i: (0, i)),
            pl.BlockSpec((H, 4 * H), lambda i: (0, 0)),
        ],
        out_specs=pl.BlockSpec((_CBLK, 2 * H), lambda i: (i, 0)),
        out_shape=jax.ShapeDtypeStruct((_ROWS_PAD, 2 * H), jnp.uint32),
    )(table_t, wab)
    return out.reshape(4 * _ROWS_PAD, _PW)


_G = 8
_NW = 32
_GROUPS = B // _G
_GPW = _GROUPS // _NW
_OUTBUF_GROUPS = 8


def _sc_mesh():
    return plsc.VectorSubcoreMesh(core_axis_name="c", subcore_axis_name="s")


@functools.partial(
    pl.kernel,
    out_type=jax.ShapeDtypeStruct((B, _PW), jnp.uint32),
    mesh=_sc_mesh(),
    compiler_params=pltpu.CompilerParams(use_tc_tiling_on_sc=False),
    scratch_types=[
        pltpu.VMEM((2, _G * L), jnp.int32),
        pltpu.VMEM((2, _G * L, _PW), jnp.uint32),
        pltpu.VMEM((_OUTBUF_GROUPS * _G, _PW), jnp.uint32),
        pltpu.SemaphoreType.DMA,
        pltpu.SemaphoreType.DMA,
        pltpu.SemaphoreType.DMA,
        pltpu.SemaphoreType.DMA,
    ],
)
def _sc_gather_max(x_hbm, table2_hbm, out_hbm,
                   idx_v, rows_v, out_v,
                   isem0, isem1, gsem0, gsem1):
    nc = 2
    wid = lax.axis_index("s") * nc + lax.axis_index("c")
    g_base = wid * _GPW
    g_last = g_base + _GPW - 1
    isems = (isem0, isem1)
    gsems = (gsem0, gsem1)

    def fire_idx(bb, g):
        pltpu.async_copy(x_hbm.at[pl.ds(g * (_G * L), _G * L)],
                         idx_v.at[bb, pl.ds(0, _G * L)], isems[bb])

    def wait_idx(bb):
        pltpu.make_async_copy(x_hbm.at[pl.ds(g_base * (_G * L), _G * L)],
                              idx_v.at[bb, pl.ds(0, _G * L)],
                              isems[bb]).wait()

    def _chunks(bb):
        total = _G * L
        for off in range(0, total, 128):
            n = min(128, total - off)
            yield (idx_v.at[bb, pl.ds(off, n)],
                   rows_v.at[bb, pl.ds(off, n)])

    def fire_gather(bb):
        wait_idx(bb)
        for k in range(_G * L // 16):
            sl = pl.ds(k * 16, 16)
            idx_v[bb, sl] = idx_v[bb, sl] << 2
        for idx_ref, dst in _chunks(bb):
            pltpu.async_copy(table2_hbm.at[idx_ref], dst, gsems[bb])

    def drain_gather(bb):
        for idx_ref, dst in _chunks(bb):
            pltpu.make_async_copy(table2_hbm.at[idx_ref], dst,
                                  gsems[bb]).wait()

    def reduce_store(bb, slot):
        def red_body(j, acc):
            new = []
            for s in range(_G):
                row = s * L + j
                for h in range(2):
                    v = rows_v[bb, row, pl.ds(h * 16, 16)]
                    new.append(jnp.maximum(acc[s * 4 + 2 * h], v))
                    new.append(jnp.maximum(acc[s * 4 + 2 * h + 1], v << 16))
            return tuple(new)

        zero = jnp.zeros((16,), jnp.uint32)
        init = tuple(zero for _ in range(_G * 4))
        acc = lax.fori_loop(0, L, red_body, init)
        for s in range(_G):
            row = slot * _G + s
            for h in range(2):
                hi_bits = acc[s * 4 + 2 * h]
                lo_bits = acc[s * 4 + 2 * h + 1]
                word = (hi_bits & jnp.uint32(0xFFFF0000)) | (lo_bits >> 16)
                out_v[row, pl.ds(h * 16, 16)] = word

    fire_idx(0, g_base)
    fire_idx(1, g_base + 1)
    fire_gather(0)

    def body(i2, _):
        ga = g_base + 2 * i2
        gc = jnp.minimum(ga + 2, g_last)
        gd = jnp.minimum(ga + 3, g_last)
        m = lax.rem(i2, 4)

        fire_gather(1)
        drain_gather(0)
        fire_idx(0, gc)
        reduce_store(0, m * 2)
        fire_gather(0)
        drain_gather(1)
        fire_idx(1, gd)
        reduce_store(1, m * 2 + 1)

        @pl.when(m == 3)
        def _flush():
            row0 = (ga - 6) * _G
            pltpu.sync_copy(
                out_v, out_hbm.at[pl.ds(row0, _OUTBUF_GROUPS * _G)])

        return ()

    lax.fori_loop(0, _GPW // 2, body, ())
    drain_gather(0)
    wait_idx(1)


def kernel(x, table, W, b):
    table2 = _transform_table(table, W)
    x1 = x.astype(jnp.int32).reshape(B * L)
    out_pack = _sc_gather_max(x1, table2)
    def _unkey(k16):
        bits = jnp.where(k16 >= jnp.uint32(0x8000),
                         k16 ^ jnp.uint32(0x8000),
                         k16 ^ jnp.uint32(0xFFFF))
        return lax.bitcast_convert_type(bits << 16, jnp.float32)

    flo = _unkey(out_pack & jnp.uint32(0xFFFF))
    fhi = _unkey(out_pack >> 16)
    return jnp.concatenate([flo, fhi], axis=1) + b
